# Initial kernel scaffold; baseline (speedup 1.0000x reference)
import jax
import jax.numpy as jnp
from jax.experimental import pallas as pl


def kernel(numbers, relative_vectors, edge_index, num_nodes, embed_table, W0, W1, W2, mlp_w1, mlp_b1, mlp_g, mlp_bt, mlp_w2, mlp_b2, ro_w1, ro_b1, ro_g, ro_bt, ro_w2, ro_b2):
    raise NotImplementedError("write your pallas kernel here")


# speedup vs baseline: 22.8962x; 22.8962x over previous
import functools

import jax
import jax.numpy as jnp
from jax import lax
from jax.experimental import pallas as pl
from jax.experimental.pallas import tpu as pltpu
from jax.experimental.pallas import tpu_sc as plsc

NC = 2
NS = 16
NW = NC * NS


def _build_sc_hist(N_pad, CH):
    mesh = plsc.VectorSubcoreMesh(core_axis_name="c", subcore_axis_name="s")

    @functools.partial(
        pl.kernel,
        out_type=jax.ShapeDtypeStruct((NC, N_pad), jnp.float32),
        mesh=mesh,
        scratch_types=[
            pltpu.VMEM((CH, 128), jnp.int32),
            pltpu.VMEM((128,), jnp.float32),
            pltpu.VMEM_SHARED((N_pad,), jnp.float32),
        ],
    )
    def hist(recv_hbm, zeros_hbm, ones_hbm, out_hbm, idx_v, ones_v, counts_sh):
        c = lax.axis_index("c")
        s = lax.axis_index("s")
        wid = c * NS + s

        @pl.when(s == 0)
        def _():
            pltpu.sync_copy(zeros_hbm, counts_sh)

        pltpu.sync_copy(recv_hbm.at[wid], idx_v)
        pltpu.sync_copy(ones_hbm, ones_v)
        plsc.subcore_barrier()

        def body(j, carry):
            pltpu.sync_copy(ones_v, counts_sh.at[idx_v.at[j]], add=True)
            return carry

        lax.fori_loop(0, CH, body, 0)
        plsc.subcore_barrier()

        @pl.when(s == 0)
        def _():
            pltpu.sync_copy(counts_sh, out_hbm.at[c])

    return hist


def _build_sc_gather(N_pad, CE):
    mesh = plsc.VectorSubcoreMesh(core_axis_name="c", subcore_axis_name="s")

    @functools.partial(
        pl.kernel,
        out_type=(
            jax.ShapeDtypeStruct((NW, CE), jnp.float32),
            jax.ShapeDtypeStruct((NW, CE), jnp.float32),
        ),
        mesh=mesh,
        scratch_types=[
            pltpu.VMEM((N_pad,), jnp.float32),
            pltpu.VMEM((N_pad,), jnp.float32),
            pltpu.VMEM((N_pad,), jnp.int32),
            pltpu.VMEM((CE,), jnp.int32),
            pltpu.VMEM((CE,), jnp.int32),
            pltpu.VMEM((CE,), jnp.float32),
            pltpu.VMEM((CE,), jnp.float32),
        ],
        compiler_params=pltpu.CompilerParams(needs_layout_passes=False),
    )
    def gather(counts2_hbm, numbers_hbm, recv_hbm, send_hbm, w_out, code_out,
               c0, c1, nums, ridx, sidx, wbuf, cbuf):
        c = lax.axis_index("c")
        s = lax.axis_index("s")
        wid = c * NS + s
        pltpu.sync_copy(counts2_hbm.at[0], c0)
        pltpu.sync_copy(counts2_hbm.at[1], c1)
        pltpu.sync_copy(numbers_hbm, nums)
        pltpu.sync_copy(recv_hbm.at[wid], ridx)
        pltpu.sync_copy(send_hbm.at[wid], sidx)

        def sum_body(j, carry):
            sl = pl.ds(j * 16, 16)
            c0[sl] = c0[sl] + c1[sl]
            return carry

        lax.fori_loop(0, N_pad // 16, sum_body, 0)

        def body(j, carry):
            sl = pl.ds(j * 16, 16)
            cnt = plsc.load_gather(c0, [ridx[sl]])
            wbuf[sl] = 1.0 / jnp.maximum(cnt, 1.0)
            code = plsc.load_gather(nums, [sidx[sl]])
            cbuf[sl] = code.astype(jnp.float32)
            return carry

        lax.fori_loop(0, CE // 16, body, 0)
        pltpu.sync_copy(wbuf, w_out.at[wid])
        pltpu.sync_copy(cbuf, code_out.at[wid])

    return gather


def _tc_body(NB, n_nodes, rv_ref, w_ref, code_ref, w1c, b1c, gc, btc, W2T,
             b2c, W0T, embT, ro_w1T, ro_b1c, ro_gc, ro_btc, ro_w2T, ro_b2c,
             out_ref, acc):
    i = pl.program_id(0)

    @pl.when(i == 0)
    def _():
        acc[...] = jnp.zeros((16, 1), jnp.float32)

    rv = rv_ref[...]
    x = rv[0:1, :]
    y = rv[1:2, :]
    z = rv[2:3, :]
    nsq = x * x + y * y + z * z
    n = jnp.sqrt(nsq)

    w1 = w1c[...]
    b1 = b1c[...]
    mw = jnp.mean(w1, axis=0, keepdims=True)
    mb = jnp.mean(b1, axis=0, keepdims=True)
    a = w1 - mw
    cc = b1 - mb
    Aq = jnp.mean(a * a, axis=0, keepdims=True)
    Bq = jnp.mean(a * cc, axis=0, keepdims=True)
    Cq = jnp.mean(cc * cc, axis=0, keepdims=True)
    d = lax.rsqrt(Aq * nsq + 2.0 * Bq * n + Cq + 1e-5)
    p = n * d

    g32 = gc[...]
    a2 = a * g32
    c2 = cc * g32
    h = a2 * p + c2 * d + btc[...]
    ysil = h * (1.0 / (1.0 + jnp.exp(-h)))

    Z = jnp.dot(W2T[...], ysil, preferred_element_type=jnp.float32, precision=lax.Precision.HIGHEST) + b2c[...]

    PT = jnp.dot(W0T[...], embT[...], preferred_element_type=jnp.float32, precision=lax.Precision.HIGHEST) * 0.25
    code = code_ref[...]
    iot = lax.broadcasted_iota(jnp.int32, (12, code.shape[1]), 0)
    oh = jnp.where(iot.astype(jnp.float32) == code, 1.0, 0.0)
    Pcols = jnp.dot(PT, oh, preferred_element_type=jnp.float32, precision=lax.Precision.HIGHEST)

    V = Z * Pcols * w_ref[...]
    acc[...] = acc[...] + jnp.sum(V, axis=1, keepdims=True)

    @pl.when(i == NB - 1)
    def _():
        G = acc[...] * (1.0 / float(n_nodes))
        r = jnp.dot(ro_w1T[...], G, preferred_element_type=jnp.float32, precision=lax.Precision.HIGHEST) + ro_b1c[...]
        m = jnp.mean(r, axis=0, keepdims=True)
        v = jnp.mean((r - m) ** 2, axis=0, keepdims=True)
        rh = (r - m) * lax.rsqrt(v + 1e-5) * ro_gc[...] + ro_btc[...]
        rs = rh * (1.0 / (1.0 + jnp.exp(-rh)))
        out_ref[...] = jnp.dot(ro_w2T[...], rs, preferred_element_type=jnp.float32, precision=lax.Precision.HIGHEST) + ro_b2c[...]


def _tc_forward(rvT, w_e, code_e, params, n_nodes, B=5120):
    E = rvT.shape[1]
    NB = E // B
    small = pl.BlockSpec(None, lambda i: tuple(0 for _ in range(2)))
    specs = [
        pl.BlockSpec((3, B), lambda i: (0, i)),
        pl.BlockSpec((1, B), lambda i: (0, i)),
        pl.BlockSpec((1, B), lambda i: (0, i)),
    ] + [pl.BlockSpec(p.shape, lambda i: (0, 0)) for p in params]
    return pl.pallas_call(
        functools.partial(_tc_body, NB, n_nodes),
        grid=(NB,),
        in_specs=specs,
        out_specs=pl.BlockSpec((1, 1), lambda i: (0, 0)),
        out_shape=jax.ShapeDtypeStruct((1, 1), jnp.float32),
        scratch_shapes=[pltpu.VMEM((16, 1), jnp.float32)],
        compiler_params=pltpu.CompilerParams(
            dimension_semantics=("arbitrary",)),
    )(rvT, w_e, code_e, *params)


def kernel(numbers, relative_vectors, edge_index, num_nodes, embed_table,
           W0, W1, W2, mlp_w1, mlp_b1, mlp_g, mlp_bt, mlp_w2, mlp_b2,
           ro_w1, ro_b1, ro_g, ro_bt, ro_w2, ro_b2):
    N = numbers.shape[0]
    E = relative_vectors.shape[0]
    assert E % NW == 0
    CE = E // NW
    CH = -(-CE // 128)
    CE_pad = CH * 128
    N_pad = ((N + 16) + 15) // 16 * 16

    send = edge_index[0].astype(jnp.int32)
    recv = edge_index[1].astype(jnp.int32)
    recv_blk = recv.reshape(NW, CE)
    pad = jnp.full((NW, CE_pad - CE), N, jnp.int32)
    recv_pad = jnp.concatenate([recv_blk, pad], axis=1).reshape(NW, CH, 128)

    zeros_np = jnp.zeros((N_pad,), jnp.float32)
    ones128 = jnp.ones((128,), jnp.float32)
    numbers_pad = jnp.concatenate(
        [numbers.astype(jnp.int32), jnp.zeros((N_pad - N,), jnp.int32)])

    counts2 = _build_sc_hist(N_pad, CH)(recv_pad, zeros_np, ones128)
    w_blk, code_blk = _build_sc_gather(N_pad, CE)(
        counts2, numbers_pad, recv_blk, send.reshape(NW, CE))

    rvT = relative_vectors.T
    w_e = w_blk.reshape(1, E)
    code_e = code_blk.reshape(1, E)

    params = (
        mlp_w1.reshape(32, 1),
        mlp_b1.reshape(32, 1),
        mlp_g.reshape(32, 1),
        mlp_bt.reshape(32, 1),
        mlp_w2[:, :16].T,
        mlp_b2[:16].reshape(16, 1),
        W0.T,
        embed_table.T,
        ro_w1.T,
        ro_b1.reshape(32, 1),
        ro_g.reshape(32, 1),
        ro_bt.reshape(32, 1),
        ro_w2.T,
        ro_b2.reshape(1, 1),
    )
    del num_nodes
    return _tc_forward(rvT, w_e, code_e, params, N)


# speedup vs baseline: 33.5465x; 1.4652x over previous
import functools

import jax
import jax.numpy as jnp
from jax import lax
from jax.experimental import pallas as pl
from jax.experimental.pallas import tpu as pltpu
from jax.experimental.pallas import tpu_sc as plsc

NC = 2
NS = 16
NW = NC * NS


def _build_sc_hist(N_pad, CH):
    mesh = plsc.VectorSubcoreMesh(core_axis_name="c", subcore_axis_name="s")

    @functools.partial(
        pl.kernel,
        out_type=jax.ShapeDtypeStruct((NC, N_pad), jnp.float32),
        mesh=mesh,
        scratch_types=[
            pltpu.VMEM((CH, 128), jnp.int32),
            pltpu.VMEM((128,), jnp.float32),
            pltpu.VMEM_SHARED((N_pad,), jnp.float32),
        ],
    )
    def hist(recv_hbm, zeros_hbm, ones_hbm, out_hbm, idx_v, ones_v, counts_sh):
        c = lax.axis_index("c")
        s = lax.axis_index("s")
        wid = c * NS + s

        @pl.when(s == 0)
        def _():
            pltpu.sync_copy(zeros_hbm, counts_sh)

        pltpu.sync_copy(recv_hbm.at[wid], idx_v)
        pltpu.sync_copy(ones_hbm, ones_v)
        plsc.subcore_barrier()

        def body(j, carry):
            pltpu.sync_copy(ones_v, counts_sh.at[idx_v.at[j]], add=True)
            return carry

        lax.fori_loop(0, CH, body, 0)
        plsc.subcore_barrier()

        @pl.when(s == 0)
        def _():
            pltpu.sync_copy(counts_sh, out_hbm.at[c])

    return hist


def _build_sc_gather(N_pad, CE):
    mesh = plsc.VectorSubcoreMesh(core_axis_name="c", subcore_axis_name="s")

    @functools.partial(
        pl.kernel,
        out_type=(
            jax.ShapeDtypeStruct((NW, CE), jnp.float32),
            jax.ShapeDtypeStruct((NW, CE), jnp.float32),
        ),
        mesh=mesh,
        scratch_types=[
            pltpu.VMEM((N_pad,), jnp.float32),
            pltpu.VMEM((N_pad,), jnp.float32),
            pltpu.VMEM((N_pad,), jnp.int32),
            pltpu.VMEM((CE,), jnp.int32),
            pltpu.VMEM((CE,), jnp.int32),
            pltpu.VMEM((CE,), jnp.float32),
            pltpu.VMEM((CE,), jnp.float32),
        ],
        compiler_params=pltpu.CompilerParams(needs_layout_passes=False),
    )
    def gather(counts2_hbm, numbers_hbm, recv_hbm, send_hbm, w_out, code_out,
               c0, c1, nums, ridx, sidx, wbuf, cbuf):
        c = lax.axis_index("c")
        s = lax.axis_index("s")
        wid = c * NS + s
        pltpu.sync_copy(counts2_hbm.at[0], c0)
        pltpu.sync_copy(counts2_hbm.at[1], c1)
        pltpu.sync_copy(numbers_hbm, nums)
        pltpu.sync_copy(recv_hbm.at[wid], ridx)
        pltpu.sync_copy(send_hbm.at[wid], sidx)

        def sum_body(j, carry):
            sl = pl.ds(j * 16, 16)
            c0[sl] = c0[sl] + c1[sl]
            return carry

        lax.fori_loop(0, N_pad // 16, sum_body, 0)

        def body(j, carry):
            sl = pl.ds(j * 16, 16)
            cnt = plsc.load_gather(c0, [ridx[sl]])
            wbuf[sl] = 1.0 / jnp.maximum(cnt, 1.0)
            code = plsc.load_gather(nums, [sidx[sl]])
            cbuf[sl] = code.astype(jnp.float32)
            return carry

        lax.fori_loop(0, CE // 16, body, 0)
        pltpu.sync_copy(wbuf, w_out.at[wid])
        pltpu.sync_copy(cbuf, code_out.at[wid])

    return gather


def _tc_body(NB, n_nodes, rv_ref, w_ref, code_ref, w1c, b1c, gc, btc, W2T,
             b2c, W0T, embT, ro_w1T, ro_b1c, ro_gc, ro_btc, ro_w2T, ro_b2c,
             out_ref, acc):
    i = pl.program_id(0)

    @pl.when(i == 0)
    def _():
        acc[...] = jnp.zeros((16, 1), jnp.float32)

    rv = rv_ref[...]
    x = rv[0:1, :]
    y = rv[1:2, :]
    z = rv[2:3, :]
    nsq = x * x + y * y + z * z
    n = jnp.sqrt(nsq)

    w1 = w1c[...]
    b1 = b1c[...]
    mw = jnp.mean(w1, axis=0, keepdims=True)
    mb = jnp.mean(b1, axis=0, keepdims=True)
    a = w1 - mw
    cc = b1 - mb
    Aq = jnp.mean(a * a, axis=0, keepdims=True)
    Bq = jnp.mean(a * cc, axis=0, keepdims=True)
    Cq = jnp.mean(cc * cc, axis=0, keepdims=True)
    d = lax.rsqrt(Aq * nsq + 2.0 * Bq * n + Cq + 1e-5)
    p = n * d

    g32 = gc[...]
    a2 = a * g32
    c2 = cc * g32
    h = a2 * p + c2 * d + btc[...]
    ysil = h * (1.0 / (1.0 + jnp.exp(-h)))

    dd = functools.partial(jnp.dot, preferred_element_type=jnp.float32)
    W2Tf = W2T[...]
    W2hi = W2Tf.astype(jnp.bfloat16)
    W2lo = (W2Tf - W2hi.astype(jnp.float32)).astype(jnp.bfloat16)
    yhi = ysil.astype(jnp.bfloat16)
    ylo = (ysil - yhi.astype(jnp.float32)).astype(jnp.bfloat16)
    Z = dd(W2hi, yhi) + (dd(W2hi, ylo) + dd(W2lo, yhi)) + b2c[...]

    PT = jnp.dot(W0T[...], embT[...], preferred_element_type=jnp.float32, precision=lax.Precision.HIGHEST) * 0.25
    PThi = PT.astype(jnp.bfloat16)
    PTlo = (PT - PThi.astype(jnp.float32)).astype(jnp.bfloat16)
    code = code_ref[...]
    iot = lax.broadcasted_iota(jnp.int32, (12, code.shape[1]), 0)
    oh = jnp.where(iot.astype(jnp.float32) == code,
                   1.0, 0.0).astype(jnp.bfloat16)
    Pcols = dd(PThi, oh) + dd(PTlo, oh)

    V = Z * Pcols * w_ref[...]
    acc[...] = acc[...] + jnp.sum(V, axis=1, keepdims=True)

    @pl.when(i == NB - 1)
    def _():
        G = acc[...] * (1.0 / float(n_nodes))
        r = jnp.dot(ro_w1T[...], G, preferred_element_type=jnp.float32, precision=lax.Precision.HIGHEST) + ro_b1c[...]
        m = jnp.mean(r, axis=0, keepdims=True)
        v = jnp.mean((r - m) ** 2, axis=0, keepdims=True)
        rh = (r - m) * lax.rsqrt(v + 1e-5) * ro_gc[...] + ro_btc[...]
        rs = rh * (1.0 / (1.0 + jnp.exp(-rh)))
        out_ref[...] = jnp.dot(ro_w2T[...], rs, preferred_element_type=jnp.float32, precision=lax.Precision.HIGHEST) + ro_b2c[...]


def _tc_forward(rvT, w_e, code_e, params, n_nodes, B=12800):
    E = rvT.shape[1]
    NB = E // B
    small = pl.BlockSpec(None, lambda i: tuple(0 for _ in range(2)))
    specs = [
        pl.BlockSpec((3, B), lambda i: (0, i)),
        pl.BlockSpec((1, B), lambda i: (0, i)),
        pl.BlockSpec((1, B), lambda i: (0, i)),
    ] + [pl.BlockSpec(p.shape, lambda i: (0, 0)) for p in params]
    return pl.pallas_call(
        functools.partial(_tc_body, NB, n_nodes),
        grid=(NB,),
        in_specs=specs,
        out_specs=pl.BlockSpec((1, 1), lambda i: (0, 0)),
        out_shape=jax.ShapeDtypeStruct((1, 1), jnp.float32),
        scratch_shapes=[pltpu.VMEM((16, 1), jnp.float32)],
        compiler_params=pltpu.CompilerParams(
            dimension_semantics=("arbitrary",)),
    )(rvT, w_e, code_e, *params)


def kernel(numbers, relative_vectors, edge_index, num_nodes, embed_table,
           W0, W1, W2, mlp_w1, mlp_b1, mlp_g, mlp_bt, mlp_w2, mlp_b2,
           ro_w1, ro_b1, ro_g, ro_bt, ro_w2, ro_b2):
    N = numbers.shape[0]
    E = relative_vectors.shape[0]
    assert E % NW == 0
    CE = E // NW
    CH = -(-CE // 128)
    CE_pad = CH * 128
    N_pad = ((N + 16) + 15) // 16 * 16

    send = edge_index[0].astype(jnp.int32)
    recv = edge_index[1].astype(jnp.int32)
    recv_blk = recv.reshape(NW, CE)
    pad = jnp.full((NW, CE_pad - CE), N, jnp.int32)
    recv_pad = jnp.concatenate([recv_blk, pad], axis=1).reshape(NW, CH, 128)

    zeros_np = jnp.zeros((N_pad,), jnp.float32)
    ones128 = jnp.ones((128,), jnp.float32)
    numbers_pad = jnp.concatenate(
        [numbers.astype(jnp.int32), jnp.zeros((N_pad - N,), jnp.int32)])

    counts2 = _build_sc_hist(N_pad, CH)(recv_pad, zeros_np, ones128)
    w_blk, code_blk = _build_sc_gather(N_pad, CE)(
        counts2, numbers_pad, recv_blk, send.reshape(NW, CE))

    rvT = relative_vectors.T
    w_e = w_blk.reshape(1, E)
    code_e = code_blk.reshape(1, E)

    params = (
        mlp_w1.reshape(32, 1),
        mlp_b1.reshape(32, 1),
        mlp_g.reshape(32, 1),
        mlp_bt.reshape(32, 1),
        mlp_w2[:, :16].T,
        mlp_b2[:16].reshape(16, 1),
        W0.T,
        embed_table.T,
        ro_w1.T,
        ro_b1.reshape(32, 1),
        ro_g.reshape(32, 1),
        ro_bt.reshape(32, 1),
        ro_w2.T,
        ro_b2.reshape(1, 1),
    )
    del num_nodes
    return _tc_forward(rvT, w_e, code_e, params, N)


# speedup vs baseline: 35.4373x; 1.0564x over previous
import functools

import jax
import jax.numpy as jnp
from jax import lax
from jax.experimental import pallas as pl
from jax.experimental.pallas import tpu as pltpu
from jax.experimental.pallas import tpu_sc as plsc

NC = 2
NS = 16
NW = NC * NS


def _build_sc_hist(N_pad, CH):
    mesh = plsc.VectorSubcoreMesh(core_axis_name="c", subcore_axis_name="s")
    SL = N_pad // NS

    @functools.partial(
        pl.kernel,
        out_type=jax.ShapeDtypeStruct((NC * N_pad,), jnp.float32),
        mesh=mesh,
        scratch_types=[
            pltpu.VMEM((CH, 128), jnp.int32),
            pltpu.VMEM((128,), jnp.float32),
            pltpu.VMEM((NS, SL), jnp.float32),
            pltpu.VMEM((SL,), jnp.float32),
            pltpu.VMEM_SHARED((NS * N_pad,), jnp.float32),
        ],
    )
    def hist(recv_hbm, zeros_hbm, ones_hbm, out_hbm, idx_v, ones_v, red_v,
             out_v, counts_sh):
        c = lax.axis_index("c")
        s = lax.axis_index("s")
        wid = c * NS + s

        pltpu.sync_copy(zeros_hbm, counts_sh.at[pl.ds(s * N_pad, N_pad)])
        pltpu.sync_copy(recv_hbm.at[wid], idx_v)
        pltpu.sync_copy(ones_hbm, ones_v)

        def body(j, carry):
            pltpu.sync_copy(ones_v, counts_sh.at[idx_v.at[j]], add=True)
            return carry

        lax.fori_loop(0, CH, body, 0)
        plsc.subcore_barrier()

        for k in range(NS):
            pltpu.sync_copy(counts_sh.at[pl.ds(k * N_pad + s * SL, SL)],
                            red_v.at[k])

        def rbody(t, carry):
            sl = pl.ds(t * 16, 16)
            acc = red_v[0, sl]
            for k in range(1, NS):
                acc = acc + red_v[k, sl]
            out_v[sl] = acc
            return carry

        lax.fori_loop(0, SL // 16, rbody, 0)
        pltpu.sync_copy(out_v, out_hbm.at[pl.ds(c * N_pad + s * SL, SL)])

    return hist


def _build_sc_gather(N_pad, CE):
    mesh = plsc.VectorSubcoreMesh(core_axis_name="c", subcore_axis_name="s")

    @functools.partial(
        pl.kernel,
        out_type=(
            jax.ShapeDtypeStruct((NW, CE), jnp.float32),
            jax.ShapeDtypeStruct((NW, CE), jnp.float32),
        ),
        mesh=mesh,
        scratch_types=[
            pltpu.VMEM((N_pad,), jnp.float32),
            pltpu.VMEM((N_pad,), jnp.float32),
            pltpu.VMEM((N_pad,), jnp.int32),
            pltpu.VMEM((CE,), jnp.int32),
            pltpu.VMEM((CE,), jnp.int32),
            pltpu.VMEM((CE,), jnp.float32),
            pltpu.VMEM((CE,), jnp.float32),
        ],
        compiler_params=pltpu.CompilerParams(needs_layout_passes=False),
    )
    def gather(counts2_hbm, numbers_hbm, recv_hbm, send_hbm, w_out, code_out,
               c0, c1, nums, ridx, sidx, wbuf, cbuf):
        c = lax.axis_index("c")
        s = lax.axis_index("s")
        wid = c * NS + s
        pltpu.sync_copy(counts2_hbm.at[0], c0)
        pltpu.sync_copy(counts2_hbm.at[1], c1)
        pltpu.sync_copy(numbers_hbm, nums)
        pltpu.sync_copy(recv_hbm.at[wid], ridx)
        pltpu.sync_copy(send_hbm.at[wid], sidx)

        def sum_body(j, carry):
            sl = pl.ds(j * 16, 16)
            c0[sl] = c0[sl] + c1[sl]
            return carry

        lax.fori_loop(0, N_pad // 16, sum_body, 0)

        def body(j, carry):
            sl = pl.ds(j * 16, 16)
            cnt = plsc.load_gather(c0, [ridx[sl]])
            wbuf[sl] = 1.0 / jnp.maximum(cnt, 1.0)
            code = plsc.load_gather(nums, [sidx[sl]])
            cbuf[sl] = code.astype(jnp.float32)
            return carry

        lax.fori_loop(0, CE // 16, body, 0)
        pltpu.sync_copy(wbuf, w_out.at[wid])
        pltpu.sync_copy(cbuf, code_out.at[wid])

    return gather


def _tc_body(NB, n_nodes, rv_ref, w_ref, code_ref, w1c, b1c, gc, btc, W2T,
             b2c, W0T, embT, ro_w1T, ro_b1c, ro_gc, ro_btc, ro_w2T, ro_b2c,
             out_ref, acc):
    i = pl.program_id(0)

    @pl.when(i == 0)
    def _():
        acc[...] = jnp.zeros((16, 1), jnp.float32)

    rv = rv_ref[...]
    x = rv[0:1, :]
    y = rv[1:2, :]
    z = rv[2:3, :]
    nsq = x * x + y * y + z * z
    n = jnp.sqrt(nsq)

    w1 = w1c[...]
    b1 = b1c[...]
    mw = jnp.mean(w1, axis=0, keepdims=True)
    mb = jnp.mean(b1, axis=0, keepdims=True)
    a = w1 - mw
    cc = b1 - mb
    Aq = jnp.mean(a * a, axis=0, keepdims=True)
    Bq = jnp.mean(a * cc, axis=0, keepdims=True)
    Cq = jnp.mean(cc * cc, axis=0, keepdims=True)
    d = lax.rsqrt(Aq * nsq + 2.0 * Bq * n + Cq + 1e-5)
    p = n * d

    g32 = gc[...]
    a2 = a * g32
    c2 = cc * g32
    h = a2 * p + c2 * d + btc[...]
    ysil = h * (1.0 / (1.0 + jnp.exp(-h)))

    dd = functools.partial(jnp.dot, preferred_element_type=jnp.float32)
    Z = dd(W2T[...].astype(jnp.bfloat16),
           ysil.astype(jnp.bfloat16)) + b2c[...]

    PT = dd(W0T[...].astype(jnp.bfloat16),
            embT[...].astype(jnp.bfloat16)) * 0.25
    PThi = PT.astype(jnp.bfloat16)
    PTlo = (PT - PThi.astype(jnp.float32)).astype(jnp.bfloat16)
    code = code_ref[...]
    iot = lax.broadcasted_iota(jnp.int32, (12, code.shape[1]), 0)
    oh = jnp.where(iot.astype(jnp.float32) == code,
                   1.0, 0.0).astype(jnp.bfloat16)
    Pcols = dd(PThi, oh) + dd(PTlo, oh)

    V = Z * Pcols * w_ref[...]
    acc[...] = acc[...] + jnp.sum(V, axis=1, keepdims=True)

    @pl.when(i == NB - 1)
    def _():
        G = acc[...] / jnp.float32(n_nodes)
        r = dd(ro_w1T[...].astype(jnp.bfloat16),
               G.astype(jnp.bfloat16)) + ro_b1c[...]
        m = jnp.mean(r, axis=0, keepdims=True)
        v = jnp.mean((r - m) ** 2, axis=0, keepdims=True)
        rh = (r - m) / jnp.sqrt(v + 1e-5) * ro_gc[...] + ro_btc[...]
        rs = rh * (1.0 / (1.0 + jnp.exp(-rh)))
        out_ref[...] = jnp.dot(ro_w2T[...], rs, preferred_element_type=jnp.float32, precision=lax.Precision.HIGHEST) + ro_b2c[...]


def _tc_forward(rvT, w_e, code_e, params, n_nodes, B=12800):
    E = rvT.shape[1]
    NB = E // B
    small = pl.BlockSpec(None, lambda i: tuple(0 for _ in range(2)))
    specs = [
        pl.BlockSpec((3, B), lambda i: (0, i)),
        pl.BlockSpec((1, B), lambda i: (0, i)),
        pl.BlockSpec((1, B), lambda i: (0, i)),
    ] + [pl.BlockSpec(p.shape, lambda i: (0, 0)) for p in params]
    return pl.pallas_call(
        functools.partial(_tc_body, NB, n_nodes),
        grid=(NB,),
        in_specs=specs,
        out_specs=pl.BlockSpec((1, 1), lambda i: (0, 0)),
        out_shape=jax.ShapeDtypeStruct((1, 1), jnp.float32),
        scratch_shapes=[pltpu.VMEM((16, 1), jnp.float32)],
        compiler_params=pltpu.CompilerParams(
            dimension_semantics=("arbitrary",)),
    )(rvT, w_e, code_e, *params)


def kernel(numbers, relative_vectors, edge_index, num_nodes, embed_table,
           W0, W1, W2, mlp_w1, mlp_b1, mlp_g, mlp_bt, mlp_w2, mlp_b2,
           ro_w1, ro_b1, ro_g, ro_bt, ro_w2, ro_b2):
    N = numbers.shape[0]
    E = relative_vectors.shape[0]
    assert E % NW == 0
    CE = E // NW
    CH = -(-CE // 128)
    CE_pad = CH * 128
    N_pad = -(-(N + 1) // (16 * NS)) * (16 * NS)

    send = edge_index[0].astype(jnp.int32)
    recv = edge_index[1].astype(jnp.int32)
    recv_blk = recv.reshape(NW, CE)
    bias = ((jnp.arange(NW, dtype=jnp.int32) % NS) * N_pad)[:, None]
    pad = jnp.full((NW, CE_pad - CE), N, jnp.int32)
    recv_pad = (jnp.concatenate([recv_blk, pad], axis=1) + bias
                ).reshape(NW, CH, 128)

    zeros_np = jnp.zeros((N_pad,), jnp.float32)
    ones128 = jnp.ones((128,), jnp.float32)
    numbers_pad = jnp.concatenate(
        [numbers.astype(jnp.int32), jnp.zeros((N_pad - N,), jnp.int32)])

    counts2 = _build_sc_hist(N_pad, CH)(recv_pad, zeros_np, ones128)
    counts2 = counts2.reshape(NC, N_pad)
    w_blk, code_blk = _build_sc_gather(N_pad, CE)(
        counts2, numbers_pad, recv_blk, send.reshape(NW, CE))

    rvT = relative_vectors.T
    w_e = w_blk.reshape(1, E)
    code_e = code_blk.reshape(1, E)

    params = (
        mlp_w1.reshape(32, 1),
        mlp_b1.reshape(32, 1),
        mlp_g.reshape(32, 1),
        mlp_bt.reshape(32, 1),
        mlp_w2[:, :16].T,
        mlp_b2[:16].reshape(16, 1),
        W0.T,
        embed_table.T,
        ro_w1.T,
        ro_b1.reshape(32, 1),
        ro_g.reshape(32, 1),
        ro_bt.reshape(32, 1),
        ro_w2.T,
        ro_b2.reshape(1, 1),
    )
    del num_nodes
    return _tc_forward(rvT, w_e, code_e, params, N)


# speedup vs baseline: 38.3609x; 1.0825x over previous
import functools

import jax
import jax.numpy as jnp
from jax import lax
from jax.experimental import pallas as pl
from jax.experimental.pallas import tpu as pltpu
from jax.experimental.pallas import tpu_sc as plsc

NC = 2
NS = 16
NW = NC * NS


def _build_sc_hist(N_pad, CH):
    mesh = plsc.VectorSubcoreMesh(core_axis_name="c", subcore_axis_name="s")
    SL = N_pad // NS

    @functools.partial(
        pl.kernel,
        out_type=jax.ShapeDtypeStruct((NC * N_pad,), jnp.float32),
        mesh=mesh,
        scratch_types=[
            pltpu.VMEM((CH, 128), jnp.int32),
            pltpu.VMEM((128,), jnp.float32),
            pltpu.VMEM((NS, SL), jnp.float32),
            pltpu.VMEM((SL,), jnp.float32),
            pltpu.VMEM_SHARED((NS * N_pad,), jnp.float32),
            pltpu.SemaphoreType.DMA,
        ],
    )
    def hist(recv_hbm, zeros_hbm, ones_hbm, out_hbm, idx_v, ones_v, red_v,
             out_v, counts_sh, sem):
        c = lax.axis_index("c")
        s = lax.axis_index("s")
        wid = c * NS + s

        pltpu.sync_copy(zeros_hbm, counts_sh.at[pl.ds(s * N_pad, N_pad)])
        pltpu.sync_copy(recv_hbm.at[wid], idx_v)
        pltpu.sync_copy(ones_hbm, ones_v)

        GROUP = 16
        def group_body(g, carry):
            def fire(j, c2):
                pltpu.async_copy(
                    ones_v, counts_sh.at[idx_v.at[g * GROUP + j]], sem,
                    add=True)
                return c2
            lax.fori_loop(0, GROUP, fire, 0)

            def drain(j, c2):
                pltpu.make_async_copy(
                    ones_v, counts_sh.at[idx_v.at[g * GROUP + j]], sem
                ).wait()
                return c2
            lax.fori_loop(0, GROUP, drain, 0)
            return carry

        lax.fori_loop(0, CH // GROUP, group_body, 0)
        plsc.subcore_barrier()

        for k in range(NS):
            pltpu.sync_copy(counts_sh.at[pl.ds(k * N_pad + s * SL, SL)],
                            red_v.at[k])

        def rbody(t, carry):
            sl = pl.ds(t * 16, 16)
            acc = red_v[0, sl]
            for k in range(1, NS):
                acc = acc + red_v[k, sl]
            out_v[sl] = acc
            return carry

        lax.fori_loop(0, SL // 16, rbody, 0)
        pltpu.sync_copy(out_v, out_hbm.at[pl.ds(c * N_pad + s * SL, SL)])

    return hist


def _build_sc_gather(N_pad, CE):
    mesh = plsc.VectorSubcoreMesh(core_axis_name="c", subcore_axis_name="s")

    @functools.partial(
        pl.kernel,
        out_type=(
            jax.ShapeDtypeStruct((NW, CE), jnp.float32),
            jax.ShapeDtypeStruct((NW, CE), jnp.float32),
        ),
        mesh=mesh,
        scratch_types=[
            pltpu.VMEM((N_pad,), jnp.float32),
            pltpu.VMEM((N_pad,), jnp.float32),
            pltpu.VMEM((N_pad,), jnp.int32),
            pltpu.VMEM((CE,), jnp.int32),
            pltpu.VMEM((CE,), jnp.int32),
            pltpu.VMEM((CE,), jnp.float32),
            pltpu.VMEM((CE,), jnp.float32),
        ],
        compiler_params=pltpu.CompilerParams(needs_layout_passes=False),
    )
    def gather(counts2_hbm, numbers_hbm, recv_hbm, send_hbm, w_out, code_out,
               c0, c1, nums, ridx, sidx, wbuf, cbuf):
        c = lax.axis_index("c")
        s = lax.axis_index("s")
        wid = c * NS + s
        pltpu.sync_copy(counts2_hbm.at[0], c0)
        pltpu.sync_copy(counts2_hbm.at[1], c1)
        pltpu.sync_copy(numbers_hbm, nums)
        pltpu.sync_copy(recv_hbm.at[wid], ridx)
        pltpu.sync_copy(send_hbm.at[wid], sidx)

        def sum_body(j, carry):
            sl = pl.ds(j * 16, 16)
            c0[sl] = c0[sl] + c1[sl]
            return carry

        lax.fori_loop(0, N_pad // 16, sum_body, 0)

        def body(j, carry):
            for u in range(5):
                sl = pl.ds(j * 80 + u * 16, 16)
                cnt = plsc.load_gather(c0, [ridx[sl]])
                wbuf[sl] = 1.0 / jnp.maximum(cnt, 1.0)
                code = plsc.load_gather(nums, [sidx[sl]])
                cbuf[sl] = code.astype(jnp.float32)
            return carry

        lax.fori_loop(0, CE // 80, body, 0)
        pltpu.sync_copy(wbuf, w_out.at[wid])
        pltpu.sync_copy(cbuf, code_out.at[wid])

    return gather


def _tc_body(NB, n_nodes, rv_ref, w_ref, code_ref, w1c, b1c, gc, btc, W2T,
             b2c, W0T, embT, ro_w1T, ro_b1c, ro_gc, ro_btc, ro_w2T, ro_b2c,
             out_ref, acc):
    i = pl.program_id(0)

    @pl.when(i == 0)
    def _():
        acc[...] = jnp.zeros((16, 1), jnp.float32)

    rv = rv_ref[...]
    x = rv[0:1, :]
    y = rv[1:2, :]
    z = rv[2:3, :]
    nsq = x * x + y * y + z * z
    n = jnp.sqrt(nsq)

    w1 = w1c[...]
    b1 = b1c[...]
    mw = jnp.mean(w1, axis=0, keepdims=True)
    mb = jnp.mean(b1, axis=0, keepdims=True)
    a = w1 - mw
    cc = b1 - mb
    Aq = jnp.mean(a * a, axis=0, keepdims=True)
    Bq = jnp.mean(a * cc, axis=0, keepdims=True)
    Cq = jnp.mean(cc * cc, axis=0, keepdims=True)
    d = lax.rsqrt(Aq * nsq + 2.0 * Bq * n + Cq + 1e-5)
    p = n * d

    g32 = gc[...]
    a2 = a * g32
    c2 = cc * g32
    h = a2 * p + c2 * d + btc[...]
    ysil = h * (1.0 / (1.0 + jnp.exp(-h)))

    dd = functools.partial(jnp.dot, preferred_element_type=jnp.float32)
    Z = dd(W2T[...].astype(jnp.bfloat16),
           ysil.astype(jnp.bfloat16)) + b2c[...]

    PT = dd(W0T[...].astype(jnp.bfloat16),
            embT[...].astype(jnp.bfloat16)) * 0.25
    PThi = PT.astype(jnp.bfloat16)
    PTlo = (PT - PThi.astype(jnp.float32)).astype(jnp.bfloat16)
    code = code_ref[...]
    iot = lax.broadcasted_iota(jnp.int32, (12, code.shape[1]), 0)
    oh = jnp.where(iot.astype(jnp.float32) == code,
                   1.0, 0.0).astype(jnp.bfloat16)
    Pcols = dd(PThi, oh) + dd(PTlo, oh)

    V = Z * Pcols * w_ref[...]
    acc[...] = acc[...] + jnp.sum(V, axis=1, keepdims=True)

    @pl.when(i == NB - 1)
    def _():
        G = acc[...] / jnp.float32(n_nodes)
        r = dd(ro_w1T[...].astype(jnp.bfloat16),
               G.astype(jnp.bfloat16)) + ro_b1c[...]
        m = jnp.mean(r, axis=0, keepdims=True)
        v = jnp.mean((r - m) ** 2, axis=0, keepdims=True)
        rh = (r - m) / jnp.sqrt(v + 1e-5) * ro_gc[...] + ro_btc[...]
        rs = rh * (1.0 / (1.0 + jnp.exp(-rh)))
        out_ref[...] = jnp.dot(ro_w2T[...], rs, preferred_element_type=jnp.float32, precision=lax.Precision.HIGHEST) + ro_b2c[...]


def _tc_forward(rvT, w_e, code_e, params, n_nodes, B=25600):
    E = rvT.shape[1]
    NB = E // B
    small = pl.BlockSpec(None, lambda i: tuple(0 for _ in range(2)))
    specs = [
        pl.BlockSpec((3, B), lambda i: (0, i)),
        pl.BlockSpec((1, B), lambda i: (0, i)),
        pl.BlockSpec((1, B), lambda i: (0, i)),
    ] + [pl.BlockSpec(p.shape, lambda i: (0, 0)) for p in params]
    return pl.pallas_call(
        functools.partial(_tc_body, NB, n_nodes),
        grid=(NB,),
        in_specs=specs,
        out_specs=pl.BlockSpec((1, 1), lambda i: (0, 0)),
        out_shape=jax.ShapeDtypeStruct((1, 1), jnp.float32),
        scratch_shapes=[pltpu.VMEM((16, 1), jnp.float32)],
        compiler_params=pltpu.CompilerParams(
            dimension_semantics=("arbitrary",)),
    )(rvT, w_e, code_e, *params)


def kernel(numbers, relative_vectors, edge_index, num_nodes, embed_table,
           W0, W1, W2, mlp_w1, mlp_b1, mlp_g, mlp_bt, mlp_w2, mlp_b2,
           ro_w1, ro_b1, ro_g, ro_bt, ro_w2, ro_b2):
    N = numbers.shape[0]
    E = relative_vectors.shape[0]
    assert E % NW == 0
    CE = E // NW
    CH = -(-CE // (128 * 16)) * 16
    CE_pad = CH * 128
    N_pad = -(-(N + 1) // (16 * NS)) * (16 * NS)

    send = edge_index[0].astype(jnp.int32)
    recv = edge_index[1].astype(jnp.int32)
    recv_blk = recv.reshape(NW, CE)
    bias = ((jnp.arange(NW, dtype=jnp.int32) % NS) * N_pad)[:, None]
    pad = jnp.full((NW, CE_pad - CE), N, jnp.int32)
    recv_pad = (jnp.concatenate([recv_blk, pad], axis=1) + bias
                ).reshape(NW, CH, 128)

    zeros_np = jnp.zeros((N_pad,), jnp.float32)
    ones128 = jnp.ones((128,), jnp.float32)
    numbers_pad = jnp.concatenate(
        [numbers.astype(jnp.int32), jnp.zeros((N_pad - N,), jnp.int32)])

    counts2 = _build_sc_hist(N_pad, CH)(recv_pad, zeros_np, ones128)
    counts2 = counts2.reshape(NC, N_pad)
    w_blk, code_blk = _build_sc_gather(N_pad, CE)(
        counts2, numbers_pad, recv_blk, send.reshape(NW, CE))

    rvT = relative_vectors.T
    w_e = w_blk.reshape(1, E)
    code_e = code_blk.reshape(1, E)

    params = (
        mlp_w1.reshape(32, 1),
        mlp_b1.reshape(32, 1),
        mlp_g.reshape(32, 1),
        mlp_bt.reshape(32, 1),
        mlp_w2[:, :16].T,
        mlp_b2[:16].reshape(16, 1),
        W0.T,
        embed_table.T,
        ro_w1.T,
        ro_b1.reshape(32, 1),
        ro_g.reshape(32, 1),
        ro_bt.reshape(32, 1),
        ro_w2.T,
        ro_b2.reshape(1, 1),
    )
    del num_nodes
    return _tc_forward(rvT, w_e, code_e, params, N)


# speedup vs baseline: 39.5953x; 1.0322x over previous
import functools

import jax
import jax.numpy as jnp
from jax import lax
from jax.experimental import pallas as pl
from jax.experimental.pallas import tpu as pltpu
from jax.experimental.pallas import tpu_sc as plsc

NC = 2
NS = 16
NW = NC * NS


def _build_sc_hist(N_pad, CE):
    mesh = plsc.VectorSubcoreMesh(core_axis_name="c", subcore_axis_name="s")
    SL = N_pad // NS

    @functools.partial(
        pl.kernel,
        out_type=jax.ShapeDtypeStruct((NC * N_pad,), jnp.float32),
        mesh=mesh,
        scratch_types=[
            pltpu.VMEM((CE,), jnp.int32),
            pltpu.VMEM((N_pad,), jnp.float32),
            pltpu.VMEM((NS, SL), jnp.float32),
            pltpu.VMEM((SL,), jnp.float32),
            pltpu.VMEM_SHARED((NS * N_pad,), jnp.float32),
        ],
        compiler_params=pltpu.CompilerParams(needs_layout_passes=False),
    )
    def hist(recv_hbm, zeros_hbm, out_hbm, idx_v, hist_v, red_v,
             out_v, counts_sh):
        c = lax.axis_index("c")
        s = lax.axis_index("s")
        wid = c * NS + s

        pltpu.sync_copy(zeros_hbm, hist_v)
        pltpu.sync_copy(recv_hbm.at[wid], idx_v)
        ones16 = jnp.full((16,), 1.0, jnp.float32)

        def body(j, carry):
            for u in range(5):
                sl = pl.ds(j * 80 + u * 16, 16)
                plsc.addupdate_scatter(hist_v, [idx_v[sl]], ones16)
            return carry

        lax.fori_loop(0, CE // 80, body, 0)
        pltpu.sync_copy(hist_v, counts_sh.at[pl.ds(s * N_pad, N_pad)])
        plsc.subcore_barrier()

        for k in range(NS):
            pltpu.sync_copy(counts_sh.at[pl.ds(k * N_pad + s * SL, SL)],
                            red_v.at[k])

        def rbody(t, carry):
            sl = pl.ds(t * 16, 16)
            acc = red_v[0, sl]
            for k in range(1, NS):
                acc = acc + red_v[k, sl]
            out_v[sl] = acc
            return carry

        lax.fori_loop(0, SL // 16, rbody, 0)
        pltpu.sync_copy(out_v, out_hbm.at[pl.ds(c * N_pad + s * SL, SL)])

    return hist


def _build_sc_gather(N_pad, CE):
    mesh = plsc.VectorSubcoreMesh(core_axis_name="c", subcore_axis_name="s")

    @functools.partial(
        pl.kernel,
        out_type=(
            jax.ShapeDtypeStruct((NW, CE), jnp.float32),
            jax.ShapeDtypeStruct((NW, CE), jnp.float32),
        ),
        mesh=mesh,
        scratch_types=[
            pltpu.VMEM((N_pad,), jnp.float32),
            pltpu.VMEM((N_pad,), jnp.float32),
            pltpu.VMEM((N_pad,), jnp.int32),
            pltpu.VMEM((CE,), jnp.int32),
            pltpu.VMEM((CE,), jnp.int32),
            pltpu.VMEM((CE,), jnp.float32),
            pltpu.VMEM((CE,), jnp.float32),
        ],
        compiler_params=pltpu.CompilerParams(needs_layout_passes=False),
    )
    def gather(counts2_hbm, numbers_hbm, recv_hbm, send_hbm, w_out, code_out,
               c0, c1, nums, ridx, sidx, wbuf, cbuf):
        c = lax.axis_index("c")
        s = lax.axis_index("s")
        wid = c * NS + s
        pltpu.sync_copy(counts2_hbm.at[0], c0)
        pltpu.sync_copy(counts2_hbm.at[1], c1)
        pltpu.sync_copy(numbers_hbm, nums)
        pltpu.sync_copy(recv_hbm.at[wid], ridx)
        pltpu.sync_copy(send_hbm.at[wid], sidx)

        def sum_body(j, carry):
            sl = pl.ds(j * 16, 16)
            c0[sl] = c0[sl] + c1[sl]
            return carry

        lax.fori_loop(0, N_pad // 16, sum_body, 0)

        def body(j, carry):
            for u in range(5):
                sl = pl.ds(j * 80 + u * 16, 16)
                cnt = plsc.load_gather(c0, [ridx[sl]])
                wbuf[sl] = 1.0 / jnp.maximum(cnt, 1.0)
                code = plsc.load_gather(nums, [sidx[sl]])
                cbuf[sl] = code.astype(jnp.float32)
            return carry

        lax.fori_loop(0, CE // 80, body, 0)
        pltpu.sync_copy(wbuf, w_out.at[wid])
        pltpu.sync_copy(cbuf, code_out.at[wid])

    return gather


def _tc_body(NB, n_nodes, rv_ref, w_ref, code_ref, w1c, b1c, gc, btc, W2T,
             b2c, W0T, embT, ro_w1T, ro_b1c, ro_gc, ro_btc, ro_w2T, ro_b2c,
             out_ref, acc):
    i = pl.program_id(0)

    @pl.when(i == 0)
    def _():
        acc[...] = jnp.zeros((16, 1), jnp.float32)

    rv = rv_ref[...]
    x = rv[0:1, :]
    y = rv[1:2, :]
    z = rv[2:3, :]
    nsq = x * x + y * y + z * z
    n = jnp.sqrt(nsq)

    w1 = w1c[...]
    b1 = b1c[...]
    mw = jnp.mean(w1, axis=0, keepdims=True)
    mb = jnp.mean(b1, axis=0, keepdims=True)
    a = w1 - mw
    cc = b1 - mb
    Aq = jnp.mean(a * a, axis=0, keepdims=True)
    Bq = jnp.mean(a * cc, axis=0, keepdims=True)
    Cq = jnp.mean(cc * cc, axis=0, keepdims=True)
    d = lax.rsqrt(Aq * nsq + 2.0 * Bq * n + Cq + 1e-5)
    p = n * d

    g32 = gc[...]
    a2 = a * g32
    c2 = cc * g32
    h = a2 * p + c2 * d + btc[...]
    ysil = h * (1.0 / (1.0 + jnp.exp(-h)))

    dd = functools.partial(jnp.dot, preferred_element_type=jnp.float32)
    Z = dd(W2T[...].astype(jnp.bfloat16),
           ysil.astype(jnp.bfloat16)) + b2c[...]

    PT = dd(W0T[...].astype(jnp.bfloat16),
            embT[...].astype(jnp.bfloat16)) * 0.25
    PThi = PT.astype(jnp.bfloat16)
    PTlo = (PT - PThi.astype(jnp.float32)).astype(jnp.bfloat16)
    code = code_ref[...]
    iot = lax.broadcasted_iota(jnp.int32, (12, code.shape[1]), 0)
    oh = jnp.where(iot.astype(jnp.float32) == code,
                   1.0, 0.0).astype(jnp.bfloat16)
    Pcols = dd(PThi, oh) + dd(PTlo, oh)

    V = Z * Pcols * w_ref[...]
    acc[...] = acc[...] + jnp.sum(V, axis=1, keepdims=True)

    @pl.when(i == NB - 1)
    def _():
        G = acc[...] / jnp.float32(n_nodes)
        r = dd(ro_w1T[...].astype(jnp.bfloat16),
               G.astype(jnp.bfloat16)) + ro_b1c[...]
        m = jnp.mean(r, axis=0, keepdims=True)
        v = jnp.mean((r - m) ** 2, axis=0, keepdims=True)
        rh = (r - m) / jnp.sqrt(v + 1e-5) * ro_gc[...] + ro_btc[...]
        rs = rh * (1.0 / (1.0 + jnp.exp(-rh)))
        out_ref[...] = jnp.dot(ro_w2T[...], rs, preferred_element_type=jnp.float32, precision=lax.Precision.HIGHEST) + ro_b2c[...]


def _tc_forward(rvT, w_e, code_e, params, n_nodes, B=25600):
    E = rvT.shape[1]
    NB = E // B
    small = pl.BlockSpec(None, lambda i: tuple(0 for _ in range(2)))
    specs = [
        pl.BlockSpec((3, B), lambda i: (0, i)),
        pl.BlockSpec((1, B), lambda i: (0, i)),
        pl.BlockSpec((1, B), lambda i: (0, i)),
    ] + [pl.BlockSpec(p.shape, lambda i: (0, 0)) for p in params]
    return pl.pallas_call(
        functools.partial(_tc_body, NB, n_nodes),
        grid=(NB,),
        in_specs=specs,
        out_specs=pl.BlockSpec((1, 1), lambda i: (0, 0)),
        out_shape=jax.ShapeDtypeStruct((1, 1), jnp.float32),
        scratch_shapes=[pltpu.VMEM((16, 1), jnp.float32)],
        compiler_params=pltpu.CompilerParams(
            dimension_semantics=("arbitrary",)),
    )(rvT, w_e, code_e, *params)


def kernel(numbers, relative_vectors, edge_index, num_nodes, embed_table,
           W0, W1, W2, mlp_w1, mlp_b1, mlp_g, mlp_bt, mlp_w2, mlp_b2,
           ro_w1, ro_b1, ro_g, ro_bt, ro_w2, ro_b2):
    N = numbers.shape[0]
    E = relative_vectors.shape[0]
    assert E % NW == 0
    CE = E // NW
    assert CE % 80 == 0
    N_pad = -(-N // (16 * NS)) * (16 * NS)

    send = edge_index[0].astype(jnp.int32)
    recv = edge_index[1].astype(jnp.int32)
    recv_blk = recv.reshape(NW, CE)

    zeros_np = jnp.zeros((N_pad,), jnp.float32)
    numbers_pad = jnp.concatenate(
        [numbers.astype(jnp.int32), jnp.zeros((N_pad - N,), jnp.int32)])

    counts2 = _build_sc_hist(N_pad, CE)(recv_blk, zeros_np)
    counts2 = counts2.reshape(NC, N_pad)
    w_blk, code_blk = _build_sc_gather(N_pad, CE)(
        counts2, numbers_pad, recv_blk, send.reshape(NW, CE))

    rvT = relative_vectors.T
    w_e = w_blk.reshape(1, E)
    code_e = code_blk.reshape(1, E)

    params = (
        mlp_w1.reshape(32, 1),
        mlp_b1.reshape(32, 1),
        mlp_g.reshape(32, 1),
        mlp_bt.reshape(32, 1),
        mlp_w2[:, :16].T,
        mlp_b2[:16].reshape(16, 1),
        W0.T,
        embed_table.T,
        ro_w1.T,
        ro_b1.reshape(32, 1),
        ro_g.reshape(32, 1),
        ro_bt.reshape(32, 1),
        ro_w2.T,
        ro_b2.reshape(1, 1),
    )
    del num_nodes
    return _tc_forward(rvT, w_e, code_e, params, N)


# speedup vs baseline: 40.6459x; 1.0265x over previous
import functools

import jax
import jax.numpy as jnp
from jax import lax
from jax.experimental import pallas as pl
from jax.experimental.pallas import tpu as pltpu
from jax.experimental.pallas import tpu_sc as plsc

NC = 2
NS = 16
NW = NC * NS


def _build_sc_hist(N_pad, CE):
    mesh = plsc.VectorSubcoreMesh(core_axis_name="c", subcore_axis_name="s")
    SL = N_pad // NS

    @functools.partial(
        pl.kernel,
        out_type=jax.ShapeDtypeStruct((NC * N_pad,), jnp.float32),
        mesh=mesh,
        scratch_types=[
            pltpu.VMEM((CE,), jnp.int32),
            pltpu.VMEM((N_pad,), jnp.float32),
            pltpu.VMEM((NS, SL), jnp.float32),
            pltpu.VMEM((SL,), jnp.float32),
            pltpu.VMEM_SHARED((NS * N_pad,), jnp.float32),
            pltpu.SemaphoreType.DMA,
        ],
        compiler_params=pltpu.CompilerParams(needs_layout_passes=False),
    )
    def hist(recv_hbm, zeros_hbm, out_hbm, idx_v, hist_v, red_v,
             out_v, counts_sh, sem):
        c = lax.axis_index("c")
        s = lax.axis_index("s")
        wid = c * NS + s

        cpy_z = pltpu.async_copy(zeros_hbm, hist_v, sem)
        cpy_r = pltpu.async_copy(recv_hbm.at[wid], idx_v, sem)
        cpy_z.wait()
        cpy_r.wait()
        ones16 = jnp.full((16,), 1.0, jnp.float32)

        def body(j, carry):
            for u in range(5):
                sl = pl.ds(j * 80 + u * 16, 16)
                plsc.addupdate_scatter(hist_v, [idx_v[sl]], ones16)
            return carry

        lax.fori_loop(0, CE // 80, body, 0)
        pltpu.sync_copy(hist_v, counts_sh.at[pl.ds(s * N_pad, N_pad)])
        plsc.subcore_barrier()

        for k in range(NS):
            pltpu.sync_copy(counts_sh.at[pl.ds(k * N_pad + s * SL, SL)],
                            red_v.at[k])

        def rbody(t, carry):
            sl = pl.ds(t * 16, 16)
            acc = red_v[0, sl]
            for k in range(1, NS):
                acc = acc + red_v[k, sl]
            out_v[sl] = acc
            return carry

        lax.fori_loop(0, SL // 16, rbody, 0)
        pltpu.sync_copy(out_v, out_hbm.at[pl.ds(c * N_pad + s * SL, SL)])

    return hist


def _build_sc_gather(N_pad, CE):
    mesh = plsc.VectorSubcoreMesh(core_axis_name="c", subcore_axis_name="s")

    @functools.partial(
        pl.kernel,
        out_type=(
            jax.ShapeDtypeStruct((NW, CE), jnp.float32),
            jax.ShapeDtypeStruct((NW, CE), jnp.float32),
        ),
        mesh=mesh,
        scratch_types=[
            pltpu.VMEM((N_pad,), jnp.float32),
            pltpu.VMEM((N_pad,), jnp.float32),
            pltpu.VMEM((N_pad,), jnp.int32),
            pltpu.VMEM((CE,), jnp.int32),
            pltpu.VMEM((CE,), jnp.int32),
            pltpu.VMEM((CE,), jnp.float32),
            pltpu.VMEM((CE,), jnp.float32),
            pltpu.SemaphoreType.DMA,
            pltpu.SemaphoreType.DMA,
        ],
        compiler_params=pltpu.CompilerParams(needs_layout_passes=False),
    )
    def gather(counts2_hbm, numbers_hbm, recv_hbm, send_hbm, w_out, code_out,
               c0, c1, nums, ridx, sidx, wbuf, cbuf, sem_c, sem_e):
        c = lax.axis_index("c")
        s = lax.axis_index("s")
        wid = c * NS + s
        cpy_c0 = pltpu.async_copy(counts2_hbm.at[0], c0, sem_c)
        cpy_c1 = pltpu.async_copy(counts2_hbm.at[1], c1, sem_c)
        cpy_nm = pltpu.async_copy(numbers_hbm, nums, sem_e)
        cpy_ri = pltpu.async_copy(recv_hbm.at[wid], ridx, sem_e)
        cpy_si = pltpu.async_copy(send_hbm.at[wid], sidx, sem_e)
        cpy_c0.wait()
        cpy_c1.wait()

        def sum_body(j, carry):
            sl = pl.ds(j * 16, 16)
            c0[sl] = c0[sl] + c1[sl]
            return carry

        lax.fori_loop(0, N_pad // 16, sum_body, 0)
        cpy_nm.wait()
        cpy_ri.wait()
        cpy_si.wait()

        def body(j, carry):
            for u in range(5):
                sl = pl.ds(j * 80 + u * 16, 16)
                cnt = plsc.load_gather(c0, [ridx[sl]])
                wbuf[sl] = 1.0 / jnp.maximum(cnt, 1.0)
                code = plsc.load_gather(nums, [sidx[sl]])
                cbuf[sl] = code.astype(jnp.float32)
            return carry

        lax.fori_loop(0, CE // 80, body, 0)
        pltpu.sync_copy(wbuf, w_out.at[wid])
        pltpu.sync_copy(cbuf, code_out.at[wid])

    return gather


def _tc_body(NB, n_nodes, rv_ref, w_ref, code_ref, w1c, b1c, gc, btc, W2T,
             b2c, W0T, embT, ro_w1T, ro_b1c, ro_gc, ro_btc, ro_w2T, ro_b2c,
             out_ref, acc):
    i = pl.program_id(0)

    @pl.when(i == 0)
    def _():
        acc[...] = jnp.zeros((16, 1), jnp.float32)

    rv = rv_ref[...]
    x = rv[0:1, :]
    y = rv[1:2, :]
    z = rv[2:3, :]
    nsq = x * x + y * y + z * z
    n = jnp.sqrt(nsq)

    w1 = w1c[...]
    b1 = b1c[...]
    mw = jnp.mean(w1, axis=0, keepdims=True)
    mb = jnp.mean(b1, axis=0, keepdims=True)
    a = w1 - mw
    cc = b1 - mb
    Aq = jnp.mean(a * a, axis=0, keepdims=True)
    Bq = jnp.mean(a * cc, axis=0, keepdims=True)
    Cq = jnp.mean(cc * cc, axis=0, keepdims=True)
    d = lax.rsqrt(Aq * nsq + 2.0 * Bq * n + Cq + 1e-5)
    p = n * d

    g32 = gc[...]
    a2 = a * g32
    c2 = cc * g32
    h = a2 * p + c2 * d + btc[...]
    ysil = h * (1.0 / (1.0 + jnp.exp(-h)))

    dd = functools.partial(jnp.dot, preferred_element_type=jnp.float32)
    Z = dd(W2T[...].astype(jnp.bfloat16),
           ysil.astype(jnp.bfloat16)) + b2c[...]

    PT = dd(W0T[...].astype(jnp.bfloat16),
            embT[...].astype(jnp.bfloat16)) * 0.25
    PThi = PT.astype(jnp.bfloat16)
    PTlo = (PT - PThi.astype(jnp.float32)).astype(jnp.bfloat16)
    code = code_ref[...]
    iot = lax.broadcasted_iota(jnp.int32, (12, code.shape[1]), 0)
    oh = jnp.where(iot.astype(jnp.float32) == code,
                   1.0, 0.0).astype(jnp.bfloat16)
    Pcols = dd(PThi, oh) + dd(PTlo, oh)

    V = Z * Pcols * w_ref[...]
    acc[...] = acc[...] + jnp.sum(V, axis=1, keepdims=True)

    @pl.when(i == NB - 1)
    def _():
        G = acc[...] / jnp.float32(n_nodes)
        r = dd(ro_w1T[...].astype(jnp.bfloat16),
               G.astype(jnp.bfloat16)) + ro_b1c[...]
        m = jnp.mean(r, axis=0, keepdims=True)
        v = jnp.mean((r - m) ** 2, axis=0, keepdims=True)
        rh = (r - m) / jnp.sqrt(v + 1e-5) * ro_gc[...] + ro_btc[...]
        rs = rh * (1.0 / (1.0 + jnp.exp(-rh)))
        out_ref[...] = jnp.dot(ro_w2T[...], rs, preferred_element_type=jnp.float32, precision=lax.Precision.HIGHEST) + ro_b2c[...]


def _tc_forward(rvT, w_e, code_e, params, n_nodes, B=25600):
    E = rvT.shape[1]
    NB = E // B
    small = pl.BlockSpec(None, lambda i: tuple(0 for _ in range(2)))
    specs = [
        pl.BlockSpec((3, B), lambda i: (0, i)),
        pl.BlockSpec((1, B), lambda i: (0, i)),
        pl.BlockSpec((1, B), lambda i: (0, i)),
    ] + [pl.BlockSpec(p.shape, lambda i: (0, 0)) for p in params]
    return pl.pallas_call(
        functools.partial(_tc_body, NB, n_nodes),
        grid=(NB,),
        in_specs=specs,
        out_specs=pl.BlockSpec((1, 1), lambda i: (0, 0)),
        out_shape=jax.ShapeDtypeStruct((1, 1), jnp.float32),
        scratch_shapes=[pltpu.VMEM((16, 1), jnp.float32)],
        compiler_params=pltpu.CompilerParams(
            dimension_semantics=("arbitrary",)),
    )(rvT, w_e, code_e, *params)


def kernel(numbers, relative_vectors, edge_index, num_nodes, embed_table,
           W0, W1, W2, mlp_w1, mlp_b1, mlp_g, mlp_bt, mlp_w2, mlp_b2,
           ro_w1, ro_b1, ro_g, ro_bt, ro_w2, ro_b2):
    N = numbers.shape[0]
    E = relative_vectors.shape[0]
    assert E % NW == 0
    CE = E // NW
    assert CE % 80 == 0
    N_pad = -(-N // (16 * NS)) * (16 * NS)

    send = edge_index[0].astype(jnp.int32)
    recv = edge_index[1].astype(jnp.int32)
    recv_blk = recv.reshape(NW, CE)

    zeros_np = jnp.zeros((N_pad,), jnp.float32)
    numbers_pad = jnp.concatenate(
        [numbers.astype(jnp.int32), jnp.zeros((N_pad - N,), jnp.int32)])

    counts2 = _build_sc_hist(N_pad, CE)(recv_blk, zeros_np)
    counts2 = counts2.reshape(NC, N_pad)
    w_blk, code_blk = _build_sc_gather(N_pad, CE)(
        counts2, numbers_pad, recv_blk, send.reshape(NW, CE))

    rvT = relative_vectors.T
    w_e = w_blk.reshape(1, E)
    code_e = code_blk.reshape(1, E)

    params = (
        mlp_w1.reshape(32, 1),
        mlp_b1.reshape(32, 1),
        mlp_g.reshape(32, 1),
        mlp_bt.reshape(32, 1),
        mlp_w2[:, :16].T,
        mlp_b2[:16].reshape(16, 1),
        W0.T,
        embed_table.T,
        ro_w1.T,
        ro_b1.reshape(32, 1),
        ro_g.reshape(32, 1),
        ro_bt.reshape(32, 1),
        ro_w2.T,
        ro_b2.reshape(1, 1),
    )
    del num_nodes
    return _tc_forward(rvT, w_e, code_e, params, N)
